# baseline (speedup 1.0000x reference)
import functools

import jax
import jax.numpy as jnp
from jax import lax
from jax.experimental import pallas as pl
from jax.experimental.pallas import tpu as pltpu
from jax.experimental.pallas import tpu_sc as plsc

N_NODES = 10000
N_EDGES = 320000
D_IN = 128
D_HID = 256
D_OUT = 128

NC = 2
NS = 16
NW = NC * NS
EDGES_PER_TILE = N_EDGES // NW
K = 80
G = 2
NGRP = 62
NWIN = 125
CH_G = 8
CH_E = CH_G * G * K
NCH = 8
N_PAD = 10240
ROWS_PER_TILE = N_PAD // NS
ZROWS = 16


def _chunk_len(ch):
    return EDGES_PER_TILE - ch * CH_E if ch == NCH - 1 else CH_E


def _sc_agg_kernel(x_hbm, ei_hbm, out_hbm, agg_sh, src_ch, dst_ch, rows,
                   zbuf, sem_g, sem_s, sem_i):
    c = lax.axis_index("c")
    s = lax.axis_index("s")
    wid = s * NC + c
    ebase = wid * EDGES_PER_TILE
    row_base = s * ROWS_PER_TILE

    def _ifire(ch, sync=False):
        q = (ch % 2) * CH_E
        off = ebase + ch * CH_E
        n = _chunk_len(ch)
        if sync:
            pltpu.sync_copy(ei_hbm.at[pl.ds(off, n)],
                            src_ch.at[pl.ds(q, n)])
            pltpu.sync_copy(ei_hbm.at[pl.ds(N_EDGES + off, n)],
                            dst_ch.at[pl.ds(q, n)])
        else:
            pltpu.async_copy(ei_hbm.at[pl.ds(off, n)],
                             src_ch.at[pl.ds(q, n)], sem_i)
            pltpu.async_copy(ei_hbm.at[pl.ds(N_EDGES + off, n)],
                             dst_ch.at[pl.ds(q, n)], sem_i)

    def _idrain(ch):
        q = (ch % 2) * CH_E
        n = _chunk_len(ch)
        pltpu.make_async_copy(ei_hbm.at[pl.ds(0, n)],
                              src_ch.at[pl.ds(q, n)], sem_i).wait()
        pltpu.make_async_copy(ei_hbm.at[pl.ds(0, n)],
                              dst_ch.at[pl.ds(q, n)], sem_i).wait()

    def _idx_off(w):
        ch = w // (2 * CH_G)
        return (ch % 2) * CH_E + (w - ch * 2 * CH_G) * K

    def _src_slice(w):
        return src_ch.at[pl.ds(_idx_off(w), K)]

    def _dst_slice(w):
        return dst_ch.at[pl.ds(_idx_off(w), K)]

    def _rows(w):
        if w == NWIN - 1:
            return rows.at[0, 0]
        return rows.at[(w // G) % 2, w % G]

    def _gfire(w):
        pltpu.async_copy(x_hbm.at[_src_slice(w)], _rows(w), sem_g)

    def _gdrain(w):
        pltpu.make_async_copy(x_hbm.at[_src_slice(w)], _rows(w),
                              sem_g).wait()

    def _sfire(w):
        pltpu.async_copy(_rows(w), agg_sh.at[_dst_slice(w)], sem_s,
                         add=True)

    def _sdrain(w):
        pltpu.make_async_copy(_rows(w), agg_sh.at[_dst_slice(w)],
                              sem_s).wait()

    def _gfire_grp(i):
        for b in range(G):
            _gfire(i * G + b)

    def _gdrain_grp(i):
        for b in range(G):
            _gdrain(i * G + b)

    def _sfire_grp(i):
        for b in range(G):
            _sfire(i * G + b)

    def _sdrain_grp(i):
        for b in range(G):
            _sdrain(i * G + b)

    _ifire(0, sync=True)
    _gfire_grp(0)
    _ifire(1)

    for r in range(ZROWS):
        for j in range(8):
            zbuf[r, pl.ds(j * 16, 16)] = jnp.zeros((16,), jnp.float32)
    for z in range(ROWS_PER_TILE // ZROWS):
        pltpu.async_copy(zbuf, agg_sh.at[pl.ds(row_base + z * ZROWS, ZROWS)],
                         sem_s)
    for z in range(ROWS_PER_TILE // ZROWS):
        pltpu.make_async_copy(zbuf,
                              agg_sh.at[pl.ds(row_base + z * ZROWS, ZROWS)],
                              sem_s).wait()

    plsc.subcore_barrier()

    for i in range(NGRP):
        _gdrain_grp(i)
        if i >= 1:
            _sdrain_grp(i - 1)
        if i > 0 and i % CH_G == 0 and i // CH_G + 1 < NCH:
            _ifire(i // CH_G + 1)
        if i + 1 < NGRP:
            if (i + 1) % CH_G == 0:
                _idrain((i + 1) // CH_G)
            _gfire_grp(i + 1)
        else:
            _gfire(NWIN - 1)
        _sfire_grp(i)
    _sdrain_grp(NGRP - 1)
    _gdrain(NWIN - 1)
    _sfire(NWIN - 1)
    _sdrain(NWIN - 1)

    plsc.subcore_barrier()

    pltpu.sync_copy(agg_sh.at[pl.ds(row_base, ROWS_PER_TILE)],
                    out_hbm.at[c, pl.ds(row_base, ROWS_PER_TILE)])


def _sc_agg(x, edge_index):
    mesh = plsc.VectorSubcoreMesh(core_axis_name="c", subcore_axis_name="s")
    return pl.kernel(
        _sc_agg_kernel,
        out_type=jax.ShapeDtypeStruct((NC, N_PAD, D_IN), jnp.float32),
        mesh=mesh,
        scratch_types=[
            pltpu.VMEM_SHARED((N_PAD, D_IN), jnp.float32),
            pltpu.VMEM((2 * CH_E,), jnp.int32),
            pltpu.VMEM((2 * CH_E,), jnp.int32),
            pltpu.VMEM((2, G, K, D_IN), jnp.float32),
            pltpu.VMEM((ZROWS, D_IN), jnp.float32),
            pltpu.SemaphoreType.DMA,
            pltpu.SemaphoreType.DMA,
            pltpu.SemaphoreType.DMA,
        ],
    )(x, edge_index.reshape(2 * N_EDGES))


def _mlp_kernel(eps_ref, x_ref, a0_ref, a1_ref, w1_ref, b1_ref, w2_ref,
                b2_ref, o_ref):
    scale = 1.0 + eps_ref[0]
    h = scale * x_ref[...] + a0_ref[...] + a1_ref[...]
    h = jnp.maximum(
        jnp.dot(h, w1_ref[...], preferred_element_type=jnp.float32)
        + b1_ref[...], 0.0)
    o_ref[...] = (
        jnp.dot(h, w2_ref[...], preferred_element_type=jnp.float32)
        + b2_ref[...])


BN = 2000


def _mlp(eps, x, agg0, agg1, W1, b1, W2, b2):
    grid = (N_NODES // BN,)
    return pl.pallas_call(
        _mlp_kernel,
        grid=grid,
        in_specs=[
            pl.BlockSpec(memory_space=pltpu.SMEM),
            pl.BlockSpec((BN, D_IN), lambda i: (i, 0)),
            pl.BlockSpec((BN, D_IN), lambda i: (i, 0)),
            pl.BlockSpec((BN, D_IN), lambda i: (i, 0)),
            pl.BlockSpec((D_IN, D_HID), lambda i: (0, 0)),
            pl.BlockSpec((1, D_HID), lambda i: (0, 0)),
            pl.BlockSpec((D_HID, D_OUT), lambda i: (0, 0)),
            pl.BlockSpec((1, D_OUT), lambda i: (0, 0)),
        ],
        out_specs=pl.BlockSpec((BN, D_OUT), lambda i: (i, 0)),
        out_shape=jax.ShapeDtypeStruct((N_NODES, D_OUT), jnp.float32),
    )(eps, x, agg0, agg1, W1, b1.reshape(1, D_HID), W2, b2.reshape(1, D_OUT))


@jax.jit
def kernel(x, edge_index, eps, W1, b1, W2, b2):
    agg = _sc_agg(x, edge_index)
    return _mlp(eps, x, agg[0], agg[1], W1, b1, W2, b2)


# speedup vs baseline: 1.1682x; 1.1682x over previous
import functools

import jax
import jax.numpy as jnp
from jax import lax
from jax.experimental import pallas as pl
from jax.experimental.pallas import tpu as pltpu
from jax.experimental.pallas import tpu_sc as plsc

N_NODES = 10000
N_EDGES = 320000
D_IN = 128
D_HID = 256
D_OUT = 128

NC = 2
NS = 16
NW = NC * NS
EDGES_PER_TILE = N_EDGES // NW
K = 80
G = 2
NGRP = 62
NWIN = 125
CH_G = 8
CH_E = CH_G * G * K
NCH = 8
N_PAD = 10240
ROWS_PER_TILE = N_PAD // NS
ZROWS = 16


def _chunk_len(ch):
    return EDGES_PER_TILE - ch * CH_E if ch == NCH - 1 else CH_E


def _sc_agg_kernel(x_hbm, ei_hbm, out_hbm, agg_sh, src_ch, dst_ch, rows,
                   zbuf, sem_g0, sem_g1, sem_s0, sem_s1, sem_i):
    sem_g = (sem_g0, sem_g1)
    sem_s = (sem_s0, sem_s1)
    c = lax.axis_index("c")
    s = lax.axis_index("s")
    wid = s * NC + c
    ebase = wid * EDGES_PER_TILE
    row_base = s * ROWS_PER_TILE

    def _ifire(ch, sync=False):
        q = (ch % 2) * CH_E
        off = ebase + ch * CH_E
        n = _chunk_len(ch)
        if sync:
            pltpu.sync_copy(ei_hbm.at[pl.ds(off, n)],
                            src_ch.at[pl.ds(q, n)])
            pltpu.sync_copy(ei_hbm.at[pl.ds(N_EDGES + off, n)],
                            dst_ch.at[pl.ds(q, n)])
        else:
            pltpu.async_copy(ei_hbm.at[pl.ds(off, n)],
                             src_ch.at[pl.ds(q, n)], sem_i)
            pltpu.async_copy(ei_hbm.at[pl.ds(N_EDGES + off, n)],
                             dst_ch.at[pl.ds(q, n)], sem_i)

    def _idrain(ch):
        q = (ch % 2) * CH_E
        n = _chunk_len(ch)
        pltpu.make_async_copy(ei_hbm.at[pl.ds(0, n)],
                              src_ch.at[pl.ds(q, n)], sem_i).wait()
        pltpu.make_async_copy(ei_hbm.at[pl.ds(0, n)],
                              dst_ch.at[pl.ds(q, n)], sem_i).wait()

    def _idx_off(w):
        ch = w // (2 * CH_G)
        return (ch % 2) * CH_E + (w - ch * 2 * CH_G) * K

    def _src_slice(w):
        return src_ch.at[pl.ds(_idx_off(w), K)]

    def _dst_slice(w):
        return dst_ch.at[pl.ds(_idx_off(w), K)]

    def _rows(w):
        if w == NWIN - 1:
            return rows.at[0, 0]
        return rows.at[(w // G) % 2, w % G]

    def _gfire(w):
        pltpu.async_copy(x_hbm.at[_src_slice(w)], _rows(w),
                         sem_g[(w // G) % 2])

    def _gdrain(w):
        pltpu.make_async_copy(x_hbm.at[_src_slice(w)], _rows(w),
                              sem_g[(w // G) % 2]).wait()

    def _sfire(w):
        pltpu.async_copy(_rows(w), agg_sh.at[_dst_slice(w)],
                         sem_s[(w // G) % 2], add=True)

    def _sdrain(w):
        pltpu.make_async_copy(_rows(w), agg_sh.at[_dst_slice(w)],
                              sem_s[(w // G) % 2]).wait()

    def _gfire_grp(i):
        for b in range(G):
            _gfire(i * G + b)

    def _gdrain_grp(i):
        for b in range(G):
            _gdrain(i * G + b)

    def _sfire_grp(i):
        for b in range(G):
            _sfire(i * G + b)

    def _sdrain_grp(i):
        for b in range(G):
            _sdrain(i * G + b)

    _ifire(0, sync=True)
    _gfire_grp(0)
    _ifire(1)

    for r in range(ZROWS):
        for j in range(8):
            zbuf[r, pl.ds(j * 16, 16)] = jnp.zeros((16,), jnp.float32)
    for z in range(ROWS_PER_TILE // ZROWS):
        pltpu.async_copy(zbuf, agg_sh.at[pl.ds(row_base + z * ZROWS, ZROWS)],
                         sem_s0)
    for z in range(ROWS_PER_TILE // ZROWS):
        pltpu.make_async_copy(zbuf,
                              agg_sh.at[pl.ds(row_base + z * ZROWS, ZROWS)],
                              sem_s0).wait()

    plsc.subcore_barrier()

    for i in range(NGRP):
        if i >= 1:
            _sdrain_grp(i - 1)
        if i > 0 and i % CH_G == 0 and i // CH_G + 1 < NCH:
            _ifire(i // CH_G + 1)
        if i + 1 < NGRP:
            if (i + 1) % CH_G == 0:
                _idrain((i + 1) // CH_G)
            _gfire_grp(i + 1)
        else:
            _gfire(NWIN - 1)
        _gdrain_grp(i)
        _sfire_grp(i)
    _sdrain_grp(NGRP - 1)
    _gdrain(NWIN - 1)
    _sfire(NWIN - 1)
    _sdrain(NWIN - 1)

    plsc.subcore_barrier()

    pltpu.sync_copy(agg_sh.at[pl.ds(row_base, ROWS_PER_TILE)],
                    out_hbm.at[c, pl.ds(row_base, ROWS_PER_TILE)])


def _sc_agg(x, edge_index):
    mesh = plsc.VectorSubcoreMesh(core_axis_name="c", subcore_axis_name="s")
    return pl.kernel(
        _sc_agg_kernel,
        out_type=jax.ShapeDtypeStruct((NC, N_PAD, D_IN), jnp.float32),
        mesh=mesh,
        scratch_types=[
            pltpu.VMEM_SHARED((N_PAD, D_IN), jnp.float32),
            pltpu.VMEM((2 * CH_E,), jnp.int32),
            pltpu.VMEM((2 * CH_E,), jnp.int32),
            pltpu.VMEM((2, G, K, D_IN), jnp.float32),
            pltpu.VMEM((ZROWS, D_IN), jnp.float32),
            pltpu.SemaphoreType.DMA,
            pltpu.SemaphoreType.DMA,
            pltpu.SemaphoreType.DMA,
            pltpu.SemaphoreType.DMA,
            pltpu.SemaphoreType.DMA,
        ],
    )(x, edge_index.reshape(2 * N_EDGES))


def _mlp_kernel(eps_ref, x_ref, a0_ref, a1_ref, w1_ref, b1_ref, w2_ref,
                b2_ref, o_ref):
    scale = 1.0 + eps_ref[0]
    h = scale * x_ref[...] + a0_ref[0] + a1_ref[0]
    h = jnp.maximum(
        jnp.dot(h, w1_ref[...], preferred_element_type=jnp.float32)
        + b1_ref[...], 0.0)
    o_ref[...] = (
        jnp.dot(h, w2_ref[...], preferred_element_type=jnp.float32)
        + b2_ref[...])


BN = 2000


def _mlp(eps, x, agg, W1, b1, W2, b2):
    grid = (N_NODES // BN,)
    return pl.pallas_call(
        _mlp_kernel,
        grid=grid,
        in_specs=[
            pl.BlockSpec(memory_space=pltpu.SMEM),
            pl.BlockSpec((BN, D_IN), lambda i: (i, 0)),
            pl.BlockSpec((1, BN, D_IN), lambda i: (0, i, 0)),
            pl.BlockSpec((1, BN, D_IN), lambda i: (1, i, 0)),
            pl.BlockSpec((D_IN, D_HID), lambda i: (0, 0)),
            pl.BlockSpec((1, D_HID), lambda i: (0, 0)),
            pl.BlockSpec((D_HID, D_OUT), lambda i: (0, 0)),
            pl.BlockSpec((1, D_OUT), lambda i: (0, 0)),
        ],
        out_specs=pl.BlockSpec((BN, D_OUT), lambda i: (i, 0)),
        out_shape=jax.ShapeDtypeStruct((N_NODES, D_OUT), jnp.float32),
    )(eps, x, agg, agg, W1, b1.reshape(1, D_HID), W2, b2.reshape(1, D_OUT))


@jax.jit
def kernel(x, edge_index, eps, W1, b1, W2, b2):
    agg = _sc_agg(x, edge_index)
    return _mlp(eps, x, agg, W1, b1, W2, b2)


# speedup vs baseline: 1.1746x; 1.0055x over previous
import functools

import jax
import jax.numpy as jnp
from jax import lax
from jax.experimental import pallas as pl
from jax.experimental.pallas import tpu as pltpu
from jax.experimental.pallas import tpu_sc as plsc

N_NODES = 10000
N_EDGES = 320000
D_IN = 128
D_HID = 256
D_OUT = 128

NC = 2
NS = 16
NW = NC * NS
EDGES_PER_TILE = N_EDGES // NW
K = 80
G = 2
NGRP = 62
NWIN = 125
CH_G = 8
CH_E = CH_G * G * K
NCH = 8
N_PAD = 10240
ROWS_PER_TILE = N_PAD // NS
ZROWS = 16


def _chunk_len(ch):
    return EDGES_PER_TILE - ch * CH_E if ch == NCH - 1 else CH_E


def _sc_agg_kernel(x_hbm, ei_hbm, out_hbm, agg_sh, src_ch, dst_ch, rows,
                   zbuf, sem_g0, sem_g1, sem_s0, sem_s1, sem_i):
    sem_g = (sem_g0, sem_g1)
    sem_s = (sem_s0, sem_s1)
    c = lax.axis_index("c")
    s = lax.axis_index("s")
    wid = s * NC + c
    ebase = wid * EDGES_PER_TILE
    row_base = s * ROWS_PER_TILE

    def _ifire(ch, sync=False):
        q = (ch % 2) * CH_E
        off = ebase + ch * CH_E
        n = _chunk_len(ch)
        if sync:
            pltpu.sync_copy(ei_hbm.at[pl.ds(off, n)],
                            src_ch.at[pl.ds(q, n)])
            pltpu.sync_copy(ei_hbm.at[pl.ds(N_EDGES + off, n)],
                            dst_ch.at[pl.ds(q, n)])
        else:
            pltpu.async_copy(ei_hbm.at[pl.ds(off, n)],
                             src_ch.at[pl.ds(q, n)], sem_i)
            pltpu.async_copy(ei_hbm.at[pl.ds(N_EDGES + off, n)],
                             dst_ch.at[pl.ds(q, n)], sem_i)

    def _idrain(ch):
        q = (ch % 2) * CH_E
        n = _chunk_len(ch)
        pltpu.make_async_copy(ei_hbm.at[pl.ds(0, n)],
                              src_ch.at[pl.ds(q, n)], sem_i).wait()
        pltpu.make_async_copy(ei_hbm.at[pl.ds(0, n)],
                              dst_ch.at[pl.ds(q, n)], sem_i).wait()

    def _idx_off(w):
        ch = w // (2 * CH_G)
        return (ch % 2) * CH_E + (w - ch * 2 * CH_G) * K

    def _src_slice(w):
        return src_ch.at[pl.ds(_idx_off(w), K)]

    def _dst_slice(w):
        return dst_ch.at[pl.ds(_idx_off(w), K)]

    def _rows(w):
        if w == NWIN - 1:
            return rows.at[0, 0]
        return rows.at[(w // G) % 2, w % G]

    def _gfire(w):
        pltpu.async_copy(x_hbm.at[_src_slice(w)], _rows(w),
                         sem_g[(w // G) % 2])

    def _gdrain(w):
        pltpu.make_async_copy(x_hbm.at[_src_slice(w)], _rows(w),
                              sem_g[(w // G) % 2]).wait()

    def _sfire(w):
        pltpu.async_copy(_rows(w), agg_sh.at[_dst_slice(w)],
                         sem_s[(w // G) % 2], add=True)

    def _sdrain(w):
        pltpu.make_async_copy(_rows(w), agg_sh.at[_dst_slice(w)],
                              sem_s[(w // G) % 2]).wait()

    def _gfire_grp(i):
        for b in range(G):
            _gfire(i * G + b)

    def _gdrain_grp(i):
        for b in range(G):
            _gdrain(i * G + b)

    def _sfire_grp(i):
        for b in range(G):
            _sfire(i * G + b)

    def _sdrain_grp(i):
        for b in range(G):
            _sdrain(i * G + b)

    _ifire(0)
    _ifire(1)
    _idrain(0)
    _gfire_grp(0)
    _gfire_grp(1)

    for r in range(ZROWS):
        for j in range(8):
            zbuf[r, pl.ds(j * 16, 16)] = jnp.zeros((16,), jnp.float32)
    for z in range(ROWS_PER_TILE // ZROWS):
        pltpu.async_copy(zbuf, agg_sh.at[pl.ds(row_base + z * ZROWS, ZROWS)],
                         sem_s0)
    for z in range(ROWS_PER_TILE // ZROWS):
        pltpu.make_async_copy(zbuf,
                              agg_sh.at[pl.ds(row_base + z * ZROWS, ZROWS)],
                              sem_s0).wait()

    plsc.subcore_barrier()

    for i in range(NGRP):
        if i >= 1:
            _sdrain_grp(i - 1)
        if i > 0 and i % CH_G == 0 and i // CH_G + 1 < NCH:
            _ifire(i // CH_G + 1)
        if 1 <= i + 1 < NGRP:
            if (i + 1) % CH_G == 0:
                _idrain((i + 1) // CH_G)
            if i >= 1:
                _gfire_grp(i + 1)
        if i + 1 == NGRP:
            _gfire(NWIN - 1)
        _gdrain_grp(i)
        _sfire_grp(i)
    _sdrain_grp(NGRP - 1)
    _gdrain(NWIN - 1)
    _sfire(NWIN - 1)
    _sdrain(NWIN - 1)

    plsc.subcore_barrier()

    pltpu.sync_copy(agg_sh.at[pl.ds(row_base, ROWS_PER_TILE)],
                    out_hbm.at[c, pl.ds(row_base, ROWS_PER_TILE)])


def _sc_agg(x, edge_index):
    mesh = plsc.VectorSubcoreMesh(core_axis_name="c", subcore_axis_name="s")
    return pl.kernel(
        _sc_agg_kernel,
        out_type=jax.ShapeDtypeStruct((NC, N_PAD, D_IN), jnp.float32),
        mesh=mesh,
        scratch_types=[
            pltpu.VMEM_SHARED((N_PAD, D_IN), jnp.float32),
            pltpu.VMEM((2 * CH_E,), jnp.int32),
            pltpu.VMEM((2 * CH_E,), jnp.int32),
            pltpu.VMEM((2, G, K, D_IN), jnp.float32),
            pltpu.VMEM((ZROWS, D_IN), jnp.float32),
            pltpu.SemaphoreType.DMA,
            pltpu.SemaphoreType.DMA,
            pltpu.SemaphoreType.DMA,
            pltpu.SemaphoreType.DMA,
            pltpu.SemaphoreType.DMA,
        ],
    )(x, edge_index.reshape(2 * N_EDGES))


def _mlp_kernel(eps_ref, x_ref, a0_ref, a1_ref, w1_ref, b1_ref, w2_ref,
                b2_ref, o_ref):
    scale = 1.0 + eps_ref[0]
    h = scale * x_ref[...] + a0_ref[0] + a1_ref[0]
    h = jnp.maximum(
        jnp.dot(h, w1_ref[...], preferred_element_type=jnp.float32)
        + b1_ref[...], 0.0)
    o_ref[...] = (
        jnp.dot(h, w2_ref[...], preferred_element_type=jnp.float32)
        + b2_ref[...])


BN = 2000


def _mlp(eps, x, agg, W1, b1, W2, b2):
    grid = (N_NODES // BN,)
    return pl.pallas_call(
        _mlp_kernel,
        grid=grid,
        in_specs=[
            pl.BlockSpec(memory_space=pltpu.SMEM),
            pl.BlockSpec((BN, D_IN), lambda i: (i, 0)),
            pl.BlockSpec((1, BN, D_IN), lambda i: (0, i, 0)),
            pl.BlockSpec((1, BN, D_IN), lambda i: (1, i, 0)),
            pl.BlockSpec((D_IN, D_HID), lambda i: (0, 0)),
            pl.BlockSpec((1, D_HID), lambda i: (0, 0)),
            pl.BlockSpec((D_HID, D_OUT), lambda i: (0, 0)),
            pl.BlockSpec((1, D_OUT), lambda i: (0, 0)),
        ],
        out_specs=pl.BlockSpec((BN, D_OUT), lambda i: (i, 0)),
        out_shape=jax.ShapeDtypeStruct((N_NODES, D_OUT), jnp.float32),
    )(eps, x, agg, agg, W1, b1.reshape(1, D_HID), W2, b2.reshape(1, D_OUT))


@jax.jit
def kernel(x, edge_index, eps, W1, b1, W2, b2):
    agg = _sc_agg(x, edge_index)
    return _mlp(eps, x, agg, W1, b1, W2, b2)
